# Initial kernel scaffold; baseline (speedup 1.0000x reference)
import jax
import jax.numpy as jnp
from jax.experimental import pallas as pl


def kernel(moe_output, scores, mapped_slots):
    raise NotImplementedError("write your pallas kernel here")


# speedup vs baseline: 3.0787x; 3.0787x over previous
import functools

import jax
import jax.numpy as jnp
from jax import lax
from jax.experimental import pallas as pl
from jax.experimental.pallas import tpu as pltpu
from jax.experimental.pallas import tpu_sc as plsc

TOP_K = 2
N_TOKENS = 8192
HIDDEN = 2048
N_SLOTS = N_TOKENS * TOP_K

NUM_WORKERS = 32
TOK_PER_WORKER = N_TOKENS // NUM_WORKERS
SLOT_PER_WORKER = TOP_K * TOK_PER_WORKER
CHUNK_T = 8
CHUNK_R = CHUNK_T * TOP_K
N_CHUNKS = TOK_PER_WORKER // CHUNK_T
LANES = 16
D_VECS = HIDDEN // LANES


def _sc_body(table_hbm, idx_hbm, sb_hbm, out_hbm, idx_v, sb_v, rows_v,
             out_v, sem):
    c = lax.axis_index("c")
    s = lax.axis_index("s")
    wid = s * 2 + c
    slot_base = wid * SLOT_PER_WORKER
    tok_base = wid * TOK_PER_WORKER

    pltpu.sync_copy(idx_hbm.at[pl.ds(slot_base, SLOT_PER_WORKER)], idx_v)
    pltpu.sync_copy(sb_hbm.at[pl.ds(slot_base, SLOT_PER_WORKER)], sb_v)

    def chunk_body(g, carry):
        idx_vec = idx_v[pl.ds(g * CHUNK_R, CHUNK_R)]
        cp = pltpu.async_copy(table_hbm.at[idx_vec], rows_v, sem)
        cp.wait()

        scales = [sb_v[g * CHUNK_R + r] for r in range(CHUNK_R)]

        def d_body(d, dcarry):
            col = pl.ds(d * LANES, LANES)
            for t in range(CHUNK_T):
                r0 = rows_v[2 * t, col]
                r1 = rows_v[2 * t + 1, col]
                out_v[t, col] = scales[2 * t] * r0 + scales[2 * t + 1] * r1
            return dcarry

        lax.fori_loop(0, D_VECS, d_body, 0)
        pltpu.sync_copy(out_v,
                        out_hbm.at[pl.ds(tok_base + g * CHUNK_T, CHUNK_T)])
        return carry

    lax.fori_loop(0, N_CHUNKS, chunk_body, 0)


@jax.jit
def kernel(moe_output, scores, mapped_slots):
    idx = mapped_slots.astype(jnp.int32)
    scores_b = jnp.broadcast_to(scores[:, None], (N_SLOTS, LANES))

    mesh = plsc.VectorSubcoreMesh(core_axis_name="c", subcore_axis_name="s",
                                  num_cores=2, num_subcores=16)
    run = pl.kernel(
        _sc_body,
        out_type=jax.ShapeDtypeStruct((N_TOKENS, HIDDEN), jnp.float32),
        mesh=mesh,
        scratch_types=[
            pltpu.VMEM((SLOT_PER_WORKER,), jnp.int32),
            pltpu.VMEM((SLOT_PER_WORKER, LANES), jnp.float32),
            pltpu.VMEM((CHUNK_R, HIDDEN), jnp.float32),
            pltpu.VMEM((CHUNK_T, HIDDEN), jnp.float32),
            pltpu.SemaphoreType.DMA,
        ],
    )
    return run(moe_output, idx, scores_b)
